# baseline (speedup 1.0000x reference)
import functools

import jax
import jax.numpy as jnp
from jax import lax
from jax.experimental import pallas as pl
from jax.experimental.pallas import tpu as pltpu
from jax.experimental.pallas import tpu_sc as plsc

_VOCAB = 1000000
_EMBED = 64
_BATCH = 4096
_SEQ = 200
_B = _BATCH * _SEQ

_NC = 2
_NS = 16
_NW = _NC * _NS
_BPW = _B // _NW
_CHUNK = 128
_NCHUNKS = _BPW // _CHUNK


@functools.partial(
    pl.kernel,
    mesh=plsc.VectorSubcoreMesh(core_axis_name="c", subcore_axis_name="s"),
    out_type=jax.ShapeDtypeStruct((_B, _EMBED), jnp.float32),
    scratch_types=[
        pltpu.VMEM((_CHUNK,), jnp.int32),
        pltpu.VMEM((_CHUNK, _EMBED), jnp.float32),
        pltpu.SemaphoreType.DMA,
    ],
    compiler_params=pltpu.CompilerParams(use_tc_tiling_on_sc=False),
)
def _gather_kernel(idx_hbm, table_hbm, out_hbm, idx_v, rows_v, sem):
    wid = lax.axis_index("s") * _NC + lax.axis_index("c")
    base = wid * _BPW

    def body(i, carry):
        off = base + i * _CHUNK
        pltpu.sync_copy(idx_hbm.at[pl.ds(off, _CHUNK)], idx_v)
        pltpu.async_copy(table_hbm.at[idx_v], rows_v, sem).wait()
        pltpu.sync_copy(rows_v, out_hbm.at[pl.ds(off, _CHUNK)])
        return carry

    lax.fori_loop(0, _NCHUNKS, body, 0)


def kernel(word_vector, table):
    idx = word_vector.reshape(_B).astype(jnp.int32)
    out = _gather_kernel(idx, table)
    return out.reshape(_BATCH, _SEQ, _EMBED)


# speedup vs baseline: 1.1957x; 1.1957x over previous
import functools

import jax
import jax.numpy as jnp
from jax import lax
from jax.experimental import pallas as pl
from jax.experimental.pallas import tpu as pltpu
from jax.experimental.pallas import tpu_sc as plsc

_VOCAB = 1000000
_EMBED = 64
_BATCH = 4096
_SEQ = 200
_B = _BATCH * _SEQ

_NC = 2
_NS = 16
_NW = _NC * _NS
_BPW = _B // _NW
_CHUNK = 128
_NCHUNKS = _BPW // _CHUNK
_NBUF = 4
_NOUTER = _NCHUNKS // _NBUF


@functools.partial(
    pl.kernel,
    mesh=plsc.VectorSubcoreMesh(core_axis_name="c", subcore_axis_name="s"),
    out_type=jax.ShapeDtypeStruct((_B, _EMBED), jnp.float32),
    scratch_types=[
        pltpu.VMEM((_NCHUNKS, _CHUNK), jnp.int32),
        pltpu.VMEM((_NBUF, _CHUNK, _EMBED), jnp.float32),
        pltpu.SemaphoreType.DMA,
    ],
    compiler_params=pltpu.CompilerParams(use_tc_tiling_on_sc=False),
)
def _gather_kernel(idx_hbm, table_hbm, out_hbm, idx_v, rows_v, sem_g):
    wid = lax.axis_index("s") * _NC + lax.axis_index("c")
    base = wid * _BPW

    pltpu.sync_copy(idx_hbm.at[wid], idx_v)

    def gather_start(chunk, buf):
        pltpu.async_copy(table_hbm.at[idx_v.at[chunk]], rows_v.at[buf], sem_g)

    def gather_wait(buf):
        pltpu.make_async_copy(table_hbm.at[idx_v.at[0]], rows_v.at[buf], sem_g).wait()

    for b in range(_NBUF):
        gather_start(b, b)

    def outer(go, carry):
        for b in range(_NBUF):
            g = go * _NBUF + b
            gather_wait(b)
            pltpu.sync_copy(rows_v.at[b], out_hbm.at[pl.ds(base + g * _CHUNK, _CHUNK)])

            @pl.when(go < _NOUTER - 1)
            def _():
                gather_start(g + _NBUF, b)

        return carry

    lax.fori_loop(0, _NOUTER, outer, 0)


def kernel(word_vector, table):
    idx = word_vector.reshape(_NW, _NCHUNKS, _CHUNK).astype(jnp.int32)
    out = _gather_kernel(idx, table)
    return out.reshape(_BATCH, _SEQ, _EMBED)


# speedup vs baseline: 1.1965x; 1.0007x over previous
import functools

import jax
import jax.numpy as jnp
from jax import lax
from jax.experimental import pallas as pl
from jax.experimental.pallas import tpu as pltpu
from jax.experimental.pallas import tpu_sc as plsc

_VOCAB = 1000000
_EMBED = 64
_BATCH = 4096
_SEQ = 200
_B = _BATCH * _SEQ

_NC = 2
_NS = 16
_NW = _NC * _NS
_BPW = _B // _NW
_CHUNK = 128
_NCHUNKS = _BPW // _CHUNK
_DEPTH = 4
_NBUF = 2 * _DEPTH
_NOUTER = _NCHUNKS // _NBUF


@functools.partial(
    pl.kernel,
    mesh=plsc.VectorSubcoreMesh(core_axis_name="c", subcore_axis_name="s"),
    out_type=jax.ShapeDtypeStruct((_B, _EMBED), jnp.float32),
    scratch_types=[
        pltpu.VMEM((_NCHUNKS, _CHUNK), jnp.int32),
        pltpu.VMEM((_NBUF, _CHUNK, _EMBED), jnp.float32),
        pltpu.SemaphoreType.DMA,
        pltpu.SemaphoreType.DMA,
    ],
    compiler_params=pltpu.CompilerParams(use_tc_tiling_on_sc=False),
)
def _gather_kernel(idx_hbm, table_hbm, out_hbm, idx_v, rows_v, sem_g, sem_s):
    wid = lax.axis_index("s") * _NC + lax.axis_index("c")
    base = wid * _BPW

    pltpu.sync_copy(idx_hbm.at[wid], idx_v)

    def out_slice(g):
        return out_hbm.at[pl.ds(base + g * _CHUNK, _CHUNK)]

    def gather_start(chunk, buf):
        pltpu.async_copy(table_hbm.at[idx_v.at[chunk]], rows_v.at[buf], sem_g)

    def gather_wait(buf):
        pltpu.make_async_copy(table_hbm.at[idx_v.at[0]], rows_v.at[buf], sem_g).wait()

    def store_start(chunk, buf):
        pltpu.async_copy(rows_v.at[buf], out_slice(chunk), sem_s)

    def store_wait(buf):
        pltpu.make_async_copy(rows_v.at[buf], out_slice(0), sem_s).wait()

    for b in range(_DEPTH):
        gather_start(b, b)

    def outer(go, carry):
        for b in range(_NBUF):
            g = go * _NBUF + b
            gather_wait(b)
            store_start(g, b)

            @pl.when(g + _DEPTH < _NCHUNKS)
            def _():
                b2 = (b + _DEPTH) % _NBUF

                @pl.when(g >= _DEPTH)
                def _():
                    store_wait(b2)

                gather_start(g + _DEPTH, b2)

        return carry

    lax.fori_loop(0, _NOUTER, outer, 0)

    for b in range(_NBUF):
        store_wait(b)


def kernel(word_vector, table):
    idx = word_vector.reshape(_NW, _NCHUNKS, _CHUNK).astype(jnp.int32)
    out = _gather_kernel(idx, table)
    return out.reshape(_BATCH, _SEQ, _EMBED)
